# baseline (speedup 1.0000x reference)
import functools

import jax
import jax.numpy as jnp
from jax import lax
from jax.experimental import pallas as pl
from jax.experimental.pallas import tpu as pltpu
from jax.experimental.pallas import tpu_sc as plsc

NUM_EMB = 1_000_000
EMBED_DIM = 32
BATCH = 16384
HIST = 200

NW = 32
ROWS_PER_W = BATCH // NW
CHUNK_ROWS = 64
N_CHUNKS = ROWS_PER_W // CHUNK_ROWS
IDX_PER_CHUNK = CHUNK_ROWS * HIST
IDX_ROWS = IDX_PER_CHUNK // 128
XROWS_PER_W = (BATCH * HIST) // 128 // NW

TBLOCK = 10000


def _table_body(emb_a_ref, emb_b_ref, emb_c_ref, emb_d_ref, w_ref, b_ref,
                out_ref):
    for h, ref in enumerate((emb_a_ref, emb_b_ref, emb_c_ref, emb_d_ref)):
        out_ref[0, 0, pl.ds(h * TBLOCK, TBLOCK)] = (
            lax.dot_general(
                w_ref[...],
                ref[...],
                dimension_numbers=(((1,), (1,)), ((), ())),
                preferred_element_type=jnp.float32,
            )[0]
            + b_ref[0]
        )


def _make_table(emb, w_scaled, b_scaled):
    grid = (NUM_EMB // (4 * TBLOCK),)
    return pl.pallas_call(
        _table_body,
        grid=grid,
        in_specs=[
            pl.BlockSpec((TBLOCK, EMBED_DIM), lambda i: (4 * i, 0)),
            pl.BlockSpec((TBLOCK, EMBED_DIM), lambda i: (4 * i + 1, 0)),
            pl.BlockSpec((TBLOCK, EMBED_DIM), lambda i: (4 * i + 2, 0)),
            pl.BlockSpec((TBLOCK, EMBED_DIM), lambda i: (4 * i + 3, 0)),
            pl.BlockSpec((1, EMBED_DIM), lambda i: (0, 0)),
            pl.BlockSpec(memory_space=pltpu.SMEM),
        ],
        out_specs=pl.BlockSpec((1, 1, 4 * TBLOCK), lambda i: (i, 0, 0)),
        out_shape=jax.ShapeDtypeStruct(
            (NUM_EMB // (4 * TBLOCK), 1, 4 * TBLOCK), jnp.float32
        ),
    )(emb, emb, emb, emb, w_scaled, b_scaled)


def _gather_sum(x2, p):
    mesh = plsc.VectorSubcoreMesh(core_axis_name="c", subcore_axis_name="s")

    @functools.partial(
        pl.kernel,
        out_type=jax.ShapeDtypeStruct((BATCH,), jnp.float32),
        mesh=mesh,
        compiler_params=pltpu.CompilerParams(needs_layout_passes=False),
        scratch_types=[
            pltpu.VMEM((200, 128), jnp.int32),
            pltpu.VMEM((IDX_PER_CHUNK,), jnp.float32),
            pltpu.VMEM((25000,), jnp.float32),
            pltpu.VMEM((ROWS_PER_W,), jnp.float32),
            pltpu.VMEM_SHARED((NUM_EMB,), jnp.float32),
            pltpu.SemaphoreType.DMA,
        ],
    )
    def body(x2_hbm, p_hbm, y_hbm, idx_v, vals_v, bounce_v, out_v, p_sh, sem):
        wid = lax.axis_index("c") * 16 + lax.axis_index("s")
        iota200 = lax.iota(jnp.int32, 16) * HIST

        sid = lax.axis_index("s")
        @pl.when(sid < 8)
        def _():
            for t in range(5):
                off = pl.multiple_of(sid * (NUM_EMB // 8) + t * 25000, 8)
                pltpu.sync_copy(p_hbm.at[pl.ds(off, 25000)], bounce_v)
                pltpu.sync_copy(bounce_v, p_sh.at[pl.ds(off, 25000)])
        plsc.subcore_barrier()

        def chunk_body(c, carry):
            half = lax.rem(c, 2)

            @pl.when(half == 0)
            def _():
                xrow = wid * XROWS_PER_W + (c // 2) * 200
                pltpu.sync_copy(x2_hbm.at[pl.ds(xrow, 200)], idx_v)

            def fire_j(j, carry2):
                for k in range(4):
                    r = j * 4 + k
                    pltpu.async_copy(
                        p_sh.at[idx_v.at[half * IDX_ROWS + r]],
                        vals_v.at[pl.ds(pl.multiple_of(r * 128, 128), 128)],
                        sem,
                    )
                return carry2

            lax.fori_loop(0, IDX_ROWS // 4, fire_j, 0)
            pltpu.make_async_copy(
                p_hbm.at[pl.ds(0, IDX_PER_CHUNK)], vals_v, sem
            ).wait()

            for g in range(CHUNK_ROWS // 16):
                base = g * 16 * HIST

                def red(j, acc):
                    jb = base + j * 8
                    vs = [
                        plsc.load_gather(vals_v, [iota200 + (jb + k)])
                        for k in range(8)
                    ]
                    s = ((vs[0] + vs[1]) + (vs[2] + vs[3])) + (
                        (vs[4] + vs[5]) + (vs[6] + vs[7])
                    )
                    return acc + s

                acc = lax.fori_loop(
                    0, HIST // 8, red, jnp.zeros((16,), jnp.float32)
                )
                out_v[
                    pl.ds(pl.multiple_of(c * CHUNK_ROWS + g * 16, 16), 16)
                ] = acc
            return carry

        lax.fori_loop(0, N_CHUNKS, chunk_body, 0)
        pltpu.sync_copy(
            out_v, y_hbm.at[pl.ds(pl.multiple_of(wid * ROWS_PER_W, 512), ROWS_PER_W)]
        )

    return body(x2, p)


def kernel(X, emb, W, b):
    x2 = X.astype(jnp.int32).reshape(BATCH * HIST // 128, 128)
    w_scaled = W.astype(jnp.float32).reshape(1, EMBED_DIM) * (1.0 / HIST)
    b_scaled = b.astype(jnp.float32).reshape(1) * (1.0 / HIST)
    p = _make_table(emb, w_scaled, b_scaled).reshape(NUM_EMB)
    y = _gather_sum(x2, p)
    return y.reshape(BATCH, 1)


# speedup vs baseline: 1.0011x; 1.0011x over previous
import functools

import jax
import jax.numpy as jnp
from jax import lax
from jax.experimental import pallas as pl
from jax.experimental.pallas import tpu as pltpu
from jax.experimental.pallas import tpu_sc as plsc

NUM_EMB = 1_000_000
EMBED_DIM = 32
BATCH = 16384
HIST = 200

NW = 32
ROWS_PER_W = BATCH // NW
CHUNK_ROWS = 64
N_CHUNKS = ROWS_PER_W // CHUNK_ROWS
IDX_PER_CHUNK = CHUNK_ROWS * HIST
IDX_ROWS = IDX_PER_CHUNK // 128
XROWS_PER_W = (BATCH * HIST) // 128 // NW

TBLOCK = 20000


def _table_body(emb_a_ref, emb_b_ref, w_ref, b_ref, out_ref):
    for h, ref in enumerate((emb_a_ref, emb_b_ref)):
        out_ref[0, 0, pl.ds(h * TBLOCK, TBLOCK)] = (
            lax.dot_general(
                w_ref[...],
                ref[...],
                dimension_numbers=(((1,), (1,)), ((), ())),
                preferred_element_type=jnp.float32,
            )[0]
            + b_ref[0]
        )


def _make_table(emb, w_scaled, b_scaled):
    grid = (NUM_EMB // (2 * TBLOCK),)
    return pl.pallas_call(
        _table_body,
        grid=grid,
        in_specs=[
            pl.BlockSpec((TBLOCK, EMBED_DIM), lambda i: (2 * i, 0)),
            pl.BlockSpec((TBLOCK, EMBED_DIM), lambda i: (2 * i + 1, 0)),
            pl.BlockSpec((1, EMBED_DIM), lambda i: (0, 0)),
            pl.BlockSpec(memory_space=pltpu.SMEM),
        ],
        out_specs=pl.BlockSpec((1, 1, 2 * TBLOCK), lambda i: (i, 0, 0)),
        out_shape=jax.ShapeDtypeStruct(
            (NUM_EMB // (2 * TBLOCK), 1, 2 * TBLOCK), jnp.float32
        ),
    )(emb, emb, w_scaled, b_scaled)


def _gather_sum(x2, p):
    mesh = plsc.VectorSubcoreMesh(core_axis_name="c", subcore_axis_name="s")

    @functools.partial(
        pl.kernel,
        out_type=jax.ShapeDtypeStruct((BATCH,), jnp.float32),
        mesh=mesh,
        compiler_params=pltpu.CompilerParams(needs_layout_passes=False),
        scratch_types=[
            pltpu.VMEM((200, 128), jnp.int32),
            pltpu.VMEM((IDX_PER_CHUNK,), jnp.float32),
            pltpu.VMEM((25000,), jnp.float32),
            pltpu.VMEM((ROWS_PER_W,), jnp.float32),
            pltpu.VMEM_SHARED((NUM_EMB,), jnp.float32),
            pltpu.SemaphoreType.DMA,
        ],
    )
    def body(x2_hbm, p_hbm, y_hbm, idx_v, vals_v, bounce_v, out_v, p_sh, sem):
        wid = lax.axis_index("c") * 16 + lax.axis_index("s")
        iota200 = lax.iota(jnp.int32, 16) * HIST

        sid = lax.axis_index("s")
        @pl.when(sid < 8)
        def _():
            for t in range(5):
                off = pl.multiple_of(sid * (NUM_EMB // 8) + t * 25000, 8)
                pltpu.sync_copy(p_hbm.at[pl.ds(off, 25000)], bounce_v)
                pltpu.sync_copy(bounce_v, p_sh.at[pl.ds(off, 25000)])
        plsc.subcore_barrier()

        def chunk_body(c, carry):
            half = lax.rem(c, 2)

            @pl.when(half == 0)
            def _():
                xrow = wid * XROWS_PER_W + (c // 2) * 200
                pltpu.sync_copy(x2_hbm.at[pl.ds(xrow, 200)], idx_v)

            def fire_j(j, carry2):
                for k in range(4):
                    r = j * 4 + k
                    pltpu.async_copy(
                        p_sh.at[idx_v.at[half * IDX_ROWS + r]],
                        vals_v.at[pl.ds(pl.multiple_of(r * 128, 128), 128)],
                        sem,
                    )
                return carry2

            lax.fori_loop(0, IDX_ROWS // 4, fire_j, 0)
            pltpu.make_async_copy(
                p_hbm.at[pl.ds(0, IDX_PER_CHUNK)], vals_v, sem
            ).wait()

            for g in range(CHUNK_ROWS // 16):
                base = g * 16 * HIST

                def red(j, acc):
                    jb = base + j * 8
                    vs = [
                        plsc.load_gather(vals_v, [iota200 + (jb + k)])
                        for k in range(8)
                    ]
                    s = ((vs[0] + vs[1]) + (vs[2] + vs[3])) + (
                        (vs[4] + vs[5]) + (vs[6] + vs[7])
                    )
                    return acc + s

                acc = lax.fori_loop(
                    0, HIST // 8, red, jnp.zeros((16,), jnp.float32)
                )
                out_v[
                    pl.ds(pl.multiple_of(c * CHUNK_ROWS + g * 16, 16), 16)
                ] = acc
            return carry

        lax.fori_loop(0, N_CHUNKS, chunk_body, 0)
        pltpu.sync_copy(
            out_v, y_hbm.at[pl.ds(pl.multiple_of(wid * ROWS_PER_W, 512), ROWS_PER_W)]
        )

    return body(x2, p)


def kernel(X, emb, W, b):
    x2 = X.astype(jnp.int32).reshape(BATCH * HIST // 128, 128)
    w_scaled = W.astype(jnp.float32).reshape(1, EMBED_DIM) * (1.0 / HIST)
    b_scaled = b.astype(jnp.float32).reshape(1) * (1.0 / HIST)
    p = _make_table(emb, w_scaled, b_scaled).reshape(NUM_EMB)
    y = _gather_sum(x2, p)
    return y.reshape(BATCH, 1)
